# baseline (speedup 1.0000x reference)
import math

import jax
import jax.numpy as jnp
from jax import lax
from jax.experimental import pallas as pl
from jax.experimental.pallas import tpu as pltpu
from jax.experimental.pallas import tpu_sc as plsc

_N_CHARGES = 100
_OUT_DIM = 32
_SCALE = float(0.1 / math.sqrt(2.0))

_NC = 2
_NS = 16
_NW = _NC * _NS

_B = 2000
_PAIRS = _N_CHARGES * _N_CHARGES
_PAIRS_PER_TILE = _PAIRS // _NS


def _body(charges_hbm, idxi_hbm, idxj_hbm, s_hbm, r_hbm, out_hbm, pair_hbm,
          ii_v, jj_v, ci_v, cj_v, pidx_v, out_v, s_v, r_v,
          charges_sh, sem):
    cid = lax.axis_index("c")
    sid = lax.axis_index("s")
    wid = sid * _NC + cid

    n_edges = idxi_hbm.shape[0]
    e_per_w = n_edges // _NW
    n_chunks = e_per_w // _B

    pltpu.sync_copy(s_hbm, s_v)
    pltpu.sync_copy(r_hbm, r_v)

    @pl.when(sid == 0)
    def _():
        pltpu.sync_copy(charges_hbm, charges_sh)

    def build(p_loc, c):
        p = sid * _PAIRS_PER_TILE + p_loc
        a = p // _N_CHARGES
        b = p - a * _N_CHARGES
        scale = jnp.float32(_SCALE)
        for h in range(_OUT_DIM // 16):
            sv = s_v[a, pl.ds(h * 16, 16)]
            rv = r_v[b, pl.ds(h * 16, 16)]
            out_v[p_loc, pl.ds(h * 16, 16)] = (sv + rv) * scale
        return c

    lax.fori_loop(0, _PAIRS_PER_TILE, build, 0)
    pltpu.sync_copy(
        out_v.at[pl.ds(0, _PAIRS_PER_TILE), :],
        pair_hbm.at[pl.ds(sid * _PAIRS_PER_TILE, _PAIRS_PER_TILE)],
    )
    plsc.subcore_barrier()

    base0 = wid * e_per_w

    def chunk(t, c):
        base = base0 + t * _B
        pltpu.sync_copy(idxi_hbm.at[pl.ds(base, _B)], ii_v)
        pltpu.sync_copy(idxj_hbm.at[pl.ds(base, _B)], jj_v)

        cp1 = pltpu.async_copy(charges_sh.at[ii_v], ci_v, sem)
        cp2 = pltpu.async_copy(charges_sh.at[jj_v], cj_v, sem)
        cp1.wait()
        cp2.wait()

        def pgroup(g, c2):
            off = pl.multiple_of(g * 16, 16)
            av = ci_v[pl.ds(off, 16)]
            bv = cj_v[pl.ds(off, 16)]
            pidx_v[pl.ds(off, 16)] = av * _N_CHARGES + bv
            return c2

        lax.fori_loop(0, _B // 16, pgroup, 0)

        pltpu.async_copy(pair_hbm.at[pidx_v], out_v, sem).wait()
        pltpu.sync_copy(out_v, out_hbm.at[pl.ds(base, _B)])
        return c

    lax.fori_loop(0, n_chunks, chunk, 0)


def kernel(flat_charges, nuc_nuc_idx, s_table, r_table):
    n_edges = nuc_nuc_idx.shape[1]
    assert n_edges % (_NW * _B) == 0

    mesh = plsc.VectorSubcoreMesh(core_axis_name="c", subcore_axis_name="s")
    run = pl.kernel(
        _body,
        mesh=mesh,
        compiler_params=pltpu.CompilerParams(use_tc_tiling_on_sc=False),
        out_type=(
            jax.ShapeDtypeStruct((n_edges, _OUT_DIM), jnp.float32),
            jax.ShapeDtypeStruct((_PAIRS, _OUT_DIM), jnp.float32),
        ),
        scratch_types=[
            pltpu.VMEM((_B,), jnp.int32),
            pltpu.VMEM((_B,), jnp.int32),
            pltpu.VMEM((_B,), jnp.int32),
            pltpu.VMEM((_B,), jnp.int32),
            pltpu.VMEM((_B,), jnp.int32),
            pltpu.VMEM((_B, _OUT_DIM), jnp.float32),
            pltpu.VMEM((_N_CHARGES, _OUT_DIM), jnp.float32),
            pltpu.VMEM((_N_CHARGES, _OUT_DIM), jnp.float32),
            pltpu.VMEM_SHARED((flat_charges.shape[0],), jnp.int32),
            pltpu.SemaphoreType.DMA,
        ],
    )
    out, _ = run(flat_charges, nuc_nuc_idx[0], nuc_nuc_idx[1], s_table, r_table)
    return out


# speedup vs baseline: 1.0458x; 1.0458x over previous
import math

import jax
import jax.numpy as jnp
from jax import lax
from jax.experimental import pallas as pl
from jax.experimental.pallas import tpu as pltpu
from jax.experimental.pallas import tpu_sc as plsc

_N_CHARGES = 100
_OUT_DIM = 32
_SCALE = float(0.1 / math.sqrt(2.0))

_NC = 2
_NS = 16
_NW = _NC * _NS

_B = 2000
_PAIRS = _N_CHARGES * _N_CHARGES
_PAIRS_PER_TILE = _PAIRS // _NS


def _body(charges_hbm, idxi_hbm, idxj_hbm, s_hbm, r_hbm, out_hbm,
          charges_v, ii_v, jj_v, pidx_v, out_v, s_v, r_v, pair_sh, sem):
    cid = lax.axis_index("c")
    sid = lax.axis_index("s")
    wid = sid * _NC + cid

    n_edges = idxi_hbm.shape[0]
    e_per_w = n_edges // _NW
    n_chunks = e_per_w // _B

    pltpu.sync_copy(s_hbm, s_v)
    pltpu.sync_copy(r_hbm, r_v)
    pltpu.sync_copy(charges_hbm, charges_v)

    def build(p_loc, c):
        p = sid * _PAIRS_PER_TILE + p_loc
        a = p // _N_CHARGES
        b = p - a * _N_CHARGES
        scale = jnp.float32(_SCALE)
        for h in range(_OUT_DIM // 16):
            sv = s_v[a, pl.ds(h * 16, 16)]
            rv = r_v[b, pl.ds(h * 16, 16)]
            out_v[p_loc, pl.ds(h * 16, 16)] = (sv + rv) * scale
        return c

    lax.fori_loop(0, _PAIRS_PER_TILE, build, 0)
    pltpu.sync_copy(
        out_v.at[pl.ds(0, _PAIRS_PER_TILE), :],
        pair_sh.at[pl.ds(sid * _PAIRS_PER_TILE, _PAIRS_PER_TILE), :],
    )
    plsc.subcore_barrier()

    base0 = wid * e_per_w

    def chunk(t, c):
        base = base0 + t * _B
        pltpu.sync_copy(idxi_hbm.at[pl.ds(base, _B)], ii_v)
        pltpu.sync_copy(idxj_hbm.at[pl.ds(base, _B)], jj_v)

        def pgroup(g, c2):
            off = pl.multiple_of(g * 16, 16)
            iv = ii_v[pl.ds(off, 16)]
            jv = jj_v[pl.ds(off, 16)]
            wi = plsc.load_gather(charges_v, [lax.shift_right_logical(iv, 1)])
            wj = plsc.load_gather(charges_v, [lax.shift_right_logical(jv, 1)])
            ci = lax.shift_right_logical(
                wi, lax.shift_left(iv & 1, 4)) & 0xFFFF
            cj = lax.shift_right_logical(
                wj, lax.shift_left(jv & 1, 4)) & 0xFFFF
            pidx_v[pl.ds(off, 16)] = ci * _N_CHARGES + cj
            return c2

        lax.fori_loop(0, _B // 16, pgroup, 0)

        pltpu.async_copy(pair_sh.at[pidx_v], out_v, sem).wait()
        pltpu.sync_copy(out_v, out_hbm.at[pl.ds(base, _B)])
        return c

    lax.fori_loop(0, n_chunks, chunk, 0)


def kernel(flat_charges, nuc_nuc_idx, s_table, r_table):
    n_edges = nuc_nuc_idx.shape[1]
    assert n_edges % (_NW * _B) == 0

    mesh = plsc.VectorSubcoreMesh(core_axis_name="c", subcore_axis_name="s")
    run = pl.kernel(
        _body,
        mesh=mesh,
        compiler_params=pltpu.CompilerParams(
            use_tc_tiling_on_sc=False,
            needs_layout_passes=False,
        ),
        out_type=jax.ShapeDtypeStruct((n_edges, _OUT_DIM), jnp.float32),
        scratch_types=[
            pltpu.VMEM((flat_charges.shape[0] // 2,), jnp.int32),
            pltpu.VMEM((_B,), jnp.int32),
            pltpu.VMEM((_B,), jnp.int32),
            pltpu.VMEM((_B,), jnp.int32),
            pltpu.VMEM((_B, _OUT_DIM), jnp.float32),
            pltpu.VMEM((_N_CHARGES, _OUT_DIM), jnp.float32),
            pltpu.VMEM((_N_CHARGES, _OUT_DIM), jnp.float32),
            pltpu.VMEM_SHARED((_PAIRS, _OUT_DIM), jnp.float32),
            pltpu.SemaphoreType.DMA,
        ],
    )
    c = flat_charges.astype(jnp.uint32)
    packed = (c[0::2] | (c[1::2] << 16)).astype(jnp.int32)
    return run(packed, nuc_nuc_idx[0], nuc_nuc_idx[1], s_table, r_table)
